# Initial kernel scaffold; baseline (speedup 1.0000x reference)
import jax
import jax.numpy as jnp
from jax.experimental import pallas as pl


def kernel(item_ids, cat_ids, rating_feats, text_ids, item_table, cat_table, text_table, W1, b1, W2, b2, W3, b3, W4, b4):
    raise NotImplementedError("write your pallas kernel here")


# speedup vs baseline: 4.7876x; 4.7876x over previous
import functools

import jax
import jax.numpy as jnp
from jax import lax
from jax.experimental import pallas as pl
from jax.experimental.pallas import tpu as pltpu
from jax.experimental.pallas import tpu_sc as plsc

B = 16384
C = 5
L = 50
D = 64

NC = 2
NS = 16
NW = NC * NS
BPW = B // NW
CH = 16
NCHUNK = BPW // CH


def _sc_body(item_ids, cat_flat, text_flat, item_tab, cat_tab, text_tab,
             item_out, cat_out, txt_out,
             iid_v, cid_v, tid_v, item_rows, cat_rows, txt_rows,
             cat_acc, txt_acc, sem):
    wid = lax.axis_index("s") * NC + lax.axis_index("c")

    def chunk_body(c, _):
        base = wid * BPW + c * CH
        pltpu.sync_copy(item_ids.at[pl.ds(base, CH)], iid_v)
        pltpu.sync_copy(cat_flat.at[pl.ds(base * C, CH * C)], cid_v)
        pltpu.sync_copy(text_flat.at[pl.ds(base * L, CH * L)], tid_v)
        handles = []
        handles.append(pltpu.async_copy(item_tab.at[iid_v], item_rows, sem))
        handles.append(pltpu.async_copy(cat_tab.at[cid_v], cat_rows, sem))
        for g in range(10):
            sl = pl.ds(80 * g, 80)
            handles.append(pltpu.async_copy(
                text_tab.at[tid_v.at[sl]], txt_rows.at[sl], sem))
        for h in handles:
            h.wait()

        def row_body(i, _):
            for d in range(4):
                sl = pl.ds(16 * d, 16)
                a = txt_rows[i * L, sl]
                for t in range(1, L):
                    a = a + txt_rows[i * L + t, sl]
                txt_acc[i, sl] = a
                b = cat_rows[i * C, sl]
                for t in range(1, C):
                    b = b + cat_rows[i * C + t, sl]
                cat_acc[i, sl] = b
            return 0

        lax.fori_loop(0, CH, row_body, 0)
        pltpu.sync_copy(item_rows, item_out.at[pl.ds(base, CH)])
        pltpu.sync_copy(cat_acc, cat_out.at[pl.ds(base, CH)])
        pltpu.sync_copy(txt_acc, txt_out.at[pl.ds(base, CH)])
        return 0

    lax.fori_loop(0, NCHUNK, chunk_body, 0)


_sc_gather = functools.partial(
    pl.kernel,
    out_type=(
        jax.ShapeDtypeStruct((B, D), jnp.float32),
        jax.ShapeDtypeStruct((B, D), jnp.float32),
        jax.ShapeDtypeStruct((B, D), jnp.float32),
    ),
    mesh=plsc.VectorSubcoreMesh(core_axis_name="c", subcore_axis_name="s"),
    compiler_params=pltpu.CompilerParams(use_tc_tiling_on_sc=False),
    scratch_types=(
        pltpu.VMEM((CH,), jnp.int32),
        pltpu.VMEM((CH * C,), jnp.int32),
        pltpu.VMEM((CH * L,), jnp.int32),
        pltpu.VMEM((CH, D), jnp.float32),
        pltpu.VMEM((CH * C, D), jnp.float32),
        pltpu.VMEM((CH * L, D), jnp.float32),
        pltpu.VMEM((CH, D), jnp.float32),
        pltpu.VMEM((CH, D), jnp.float32),
        pltpu.SemaphoreType.DMA,
    ),
)(_sc_body)


def _tc_body(item_e, cat_s, txt_s, cat_ids, text_ids, rating, cat0,
             W1, b1, W2, b2, W3, b3, W4, b4, out):
    cnt_c = jnp.sum((cat_ids[...] != 0).astype(jnp.float32), axis=1,
                    keepdims=True)
    corr = cat_s[...] - (C - cnt_c) * cat0[...]
    cat_vec = jnp.where(cnt_c > 0.0, corr / (cnt_c + 1e-9), 0.0)
    cnt_t = jnp.sum((text_ids[...] != 0).astype(jnp.float32), axis=1,
                    keepdims=True)
    txt_vec = txt_s[...] / (cnt_t + 1e-9)
    r1 = jnp.maximum(
        jnp.dot(rating[...], W1[...], preferred_element_type=jnp.float32)
        + b1[...], 0.0)
    rate_e = jnp.dot(r1, W2[...], preferred_element_type=jnp.float32) + b2[...]
    x = jnp.concatenate([item_e[...], cat_vec, rate_e, txt_vec], axis=-1)
    h = jnp.maximum(
        jnp.dot(x, W3[...], preferred_element_type=jnp.float32) + b3[...], 0.0)
    out[...] = jnp.dot(h, W4[...], preferred_element_type=jnp.float32) + b4[...]


def _tc_mlp(item_e, cat_s, txt_s, cat_ids, text_ids, rating, cat0,
            W1, b1, W2, b2, W3, b3, W4, b4):
    BB = 2048
    grid = (B // BB,)

    def row_block(n):
        return pl.BlockSpec((BB, None), lambda i: (i, 0))

    def full(shape):
        return pl.BlockSpec(shape, lambda i: tuple(0 for _ in shape))

    return pl.pallas_call(
        _tc_body,
        grid=grid,
        in_specs=[
            pl.BlockSpec((BB, D), lambda i: (i, 0)),
            pl.BlockSpec((BB, D), lambda i: (i, 0)),
            pl.BlockSpec((BB, D), lambda i: (i, 0)),
            pl.BlockSpec((BB, C), lambda i: (i, 0)),
            pl.BlockSpec((BB, L), lambda i: (i, 0)),
            pl.BlockSpec((BB, 2), lambda i: (i, 0)),
            full((1, D)),
            full((2, 16)), full((1, 16)),
            full((16, D)), full((1, D)),
            full((3 * D + D, 128)), full((1, 128)),
            full((128, D)), full((1, D)),
        ],
        out_specs=pl.BlockSpec((BB, D), lambda i: (i, 0)),
        out_shape=jax.ShapeDtypeStruct((B, D), jnp.float32),
    )(item_e, cat_s, txt_s, cat_ids, text_ids, rating, cat0,
      W1, b1, W2, b2, W3, b3, W4, b4)


def kernel(item_ids, cat_ids, rating_feats, text_ids, item_table, cat_table,
           text_table, W1, b1, W2, b2, W3, b3, W4, b4):
    iid = item_ids.astype(jnp.int32)
    cflat = cat_ids.astype(jnp.int32).reshape(-1)
    tflat = text_ids.astype(jnp.int32).reshape(-1)
    item_e, cat_s, txt_s = _sc_gather(
        iid, cflat, tflat, item_table, cat_table, text_table)
    return _tc_mlp(
        item_e, cat_s, txt_s, cat_ids.astype(jnp.int32),
        text_ids.astype(jnp.int32), rating_feats, cat_table[0:1],
        W1, b1.reshape(1, -1), W2, b2.reshape(1, -1),
        W3, b3.reshape(1, -1), W4, b4.reshape(1, -1))
